# baseline (speedup 1.0000x reference)
import jax
import jax.numpy as jnp
from jax import lax
from jax.experimental import pallas as pl
from jax.experimental.pallas import tpu as pltpu
from jax.experimental.pallas import tpu_sc as plsc

EMBED_DIM = 32
NUM_CORES = 2
NUM_SUBCORES = 16
NUM_WORKERS = NUM_CORES * NUM_SUBCORES
VOCAB = 1000000
EBLK = 128
NFULL = VOCAB // EBLK
ETAIL = VOCAB - NFULL * EBLK

CHUNK = 1024
NBUF = 3


TBLK = 2048


def _tc_transpose_body(embT_ref, t4_ref):
    for r in range(4):
        y = embT_ref[pl.ds(8 * r, 8), :].T
        for a in range(4):
            t4_ref[:, pl.ds(32 * a + 8 * r, 8)] = lax.slice(
                y, (a * (TBLK // 4), 0), ((a + 1) * (TBLK // 4), 8)
            )


def _gather_body(idx_hbm, table_hbm, out_hbm, idx_v, rows_v, *sems):
    gsems, ssems = sems[:NBUF], sems[NBUF:]
    n_chunks = idx_hbm.shape[0] // (NUM_WORKERS * CHUNK)
    wid = lax.axis_index("s") * NUM_CORES + lax.axis_index("c")
    base = wid * (n_chunks * CHUNK)
    gathers = [None] * n_chunks
    stores = [None] * n_chunks

    def start_gather(c):
        b = c % NBUF
        pltpu.sync_copy(idx_hbm.at[pl.ds(base + c * CHUNK, CHUNK)], idx_v.at[b])
        gathers[c] = pltpu.async_copy(
            table_hbm.at[idx_v.at[b]], rows_v.at[b], gsems[b]
        )

    start_gather(0)
    for c in range(n_chunks):
        b = c % NBUF
        if c + 1 < n_chunks:
            if c + 1 >= NBUF:
                stores[c + 1 - NBUF].wait()
            start_gather(c + 1)
        gathers[c].wait()
        stores[c] = pltpu.async_copy(
            rows_v.at[b], out_hbm.at[pl.ds(base + c * CHUNK, CHUNK)], ssems[b]
        )
    for c in range(max(0, n_chunks - NBUF), n_chunks):
        stores[c].wait()


def _embed_lookup(idx_flat, table):
    n = idx_flat.shape[0]
    mesh = plsc.VectorSubcoreMesh(core_axis_name="c", subcore_axis_name="s")
    n_blocks = (VOCAB + TBLK - 1) // TBLK
    t4 = pl.pallas_call(
        _tc_transpose_body,
        grid=(n_blocks,),
        in_specs=[pl.BlockSpec((32, TBLK), lambda j: (0, j))],
        out_specs=pl.BlockSpec((TBLK // 4, 128), lambda j: (j, 0)),
        out_shape=jax.ShapeDtypeStruct((n_blocks * TBLK // 4, 128), jnp.float32),
    )(table.T)
    t_lin = t4.reshape(n_blocks * TBLK, EMBED_DIM)
    return pl.kernel(
        _gather_body,
        out_type=jax.ShapeDtypeStruct((n, EMBED_DIM), jnp.float32),
        mesh=mesh,
        scratch_types=[
            pltpu.VMEM((NBUF, CHUNK), jnp.int32),
            pltpu.VMEM((NBUF, CHUNK, EMBED_DIM), jnp.float32),
        ]
        + [pltpu.SemaphoreType.DMA] * (2 * NBUF),
        compiler_params=pltpu.CompilerParams(use_tc_tiling_on_sc=False),
    )(idx_flat, t_lin)


def kernel(embedding_input, embedding):
    batch, hist = embedding_input.shape
    idx = embedding_input.reshape(-1).astype(jnp.int32)
    slot = (idx & ~2047) | ((idx & 511) << 2) | ((idx >> 9) & 3)
    out = _embed_lookup(slot, embedding)
    return out.reshape(batch, hist, EMBED_DIM)


# speedup vs baseline: 2.4403x; 2.4403x over previous
import jax
import jax.numpy as jnp
from jax import lax
from jax.experimental import pallas as pl
from jax.experimental.pallas import tpu as pltpu
from jax.experimental.pallas import tpu_sc as plsc

EMBED_DIM = 32
NUM_CORES = 2
NUM_SUBCORES = 16
NUM_WORKERS = NUM_CORES * NUM_SUBCORES
VOCAB = 1000000
EBLK = 128
NFULL = VOCAB // EBLK
ETAIL = VOCAB - NFULL * EBLK

CHUNK = 1024
NBUF = 3


TBLK = 4096


def _tc_transpose_body(embT_ref, t4_ref):
    y = embT_ref[...].T
    for a in range(4):
        t4_ref[:, pl.ds(32 * a, 32)] = lax.slice(
            y, (a * (TBLK // 4), 0), ((a + 1) * (TBLK // 4), 32)
        )


def _gather_body(idx_hbm, table_hbm, out_hbm, idx_v, rows_v, *sems):
    gsems, ssems = sems[:NBUF], sems[NBUF:]
    n_chunks = idx_hbm.shape[0] // (NUM_WORKERS * CHUNK)
    wid = lax.axis_index("s") * NUM_CORES + lax.axis_index("c")
    base = wid * (n_chunks * CHUNK)
    gathers = [None] * n_chunks
    stores = [None] * n_chunks

    def start_gather(c):
        b = c % NBUF
        pltpu.sync_copy(idx_hbm.at[pl.ds(base + c * CHUNK, CHUNK)], idx_v.at[b])
        gathers[c] = pltpu.async_copy(
            table_hbm.at[idx_v.at[b]], rows_v.at[b], gsems[b]
        )

    start_gather(0)
    for c in range(n_chunks):
        b = c % NBUF
        if c + 1 < n_chunks:
            if c + 1 >= NBUF:
                stores[c + 1 - NBUF].wait()
            start_gather(c + 1)
        gathers[c].wait()
        stores[c] = pltpu.async_copy(
            rows_v.at[b], out_hbm.at[pl.ds(base + c * CHUNK, CHUNK)], ssems[b]
        )
    for c in range(max(0, n_chunks - NBUF), n_chunks):
        stores[c].wait()


def _embed_lookup(idx_flat, table):
    n = idx_flat.shape[0]
    mesh = plsc.VectorSubcoreMesh(core_axis_name="c", subcore_axis_name="s")
    n_blocks = (VOCAB + TBLK - 1) // TBLK
    t4 = pl.pallas_call(
        _tc_transpose_body,
        grid=(n_blocks,),
        in_specs=[pl.BlockSpec((32, TBLK), lambda j: (0, j))],
        out_specs=pl.BlockSpec((TBLK // 4, 128), lambda j: (j, 0)),
        out_shape=jax.ShapeDtypeStruct((n_blocks * TBLK // 4, 128), jnp.float32),
    )(table.T)
    t_lin = t4.reshape(n_blocks * TBLK, EMBED_DIM)
    return pl.kernel(
        _gather_body,
        out_type=jax.ShapeDtypeStruct((n, EMBED_DIM), jnp.float32),
        mesh=mesh,
        scratch_types=[
            pltpu.VMEM((NBUF, CHUNK), jnp.int32),
            pltpu.VMEM((NBUF, CHUNK, EMBED_DIM), jnp.float32),
        ]
        + [pltpu.SemaphoreType.DMA] * (2 * NBUF),
        compiler_params=pltpu.CompilerParams(use_tc_tiling_on_sc=False),
    )(idx_flat, t_lin)


def kernel(embedding_input, embedding):
    batch, hist = embedding_input.shape
    idx = embedding_input.reshape(-1).astype(jnp.int32)
    slot = (idx & ~4095) | ((idx & 1023) << 2) | ((idx >> 10) & 3)
    out = _embed_lookup(slot, embedding)
    return out.reshape(batch, hist, EMBED_DIM)


# speedup vs baseline: 2.6530x; 1.0872x over previous
import jax
import jax.numpy as jnp
from jax import lax
from jax.experimental import pallas as pl
from jax.experimental.pallas import tpu as pltpu
from jax.experimental.pallas import tpu_sc as plsc

EMBED_DIM = 32
NUM_CORES = 2
NUM_SUBCORES = 16
NUM_WORKERS = NUM_CORES * NUM_SUBCORES
VOCAB = 1000000
EBLK = 128
NFULL = VOCAB // EBLK
ETAIL = VOCAB - NFULL * EBLK

CHUNK = 1024
NBUF = 3


TBLK = 8192


def _tc_transpose_body(embT_ref, t4_ref):
    y = embT_ref[...].T
    for a in range(4):
        t4_ref[:, pl.ds(32 * a, 32)] = lax.slice(
            y, (a * (TBLK // 4), 0), ((a + 1) * (TBLK // 4), 32)
        )


def _gather_body(idx_hbm, table_hbm, out_hbm, idx_v, rows_v, *sems):
    gsems, ssems = sems[:NBUF], sems[NBUF:]
    n_chunks = idx_hbm.shape[0] // (NUM_WORKERS * CHUNK)
    wid = lax.axis_index("s") * NUM_CORES + lax.axis_index("c")
    base = wid * (n_chunks * CHUNK)
    gathers = [None] * n_chunks
    stores = [None] * n_chunks

    def start_gather(c):
        b = c % NBUF
        pltpu.sync_copy(idx_hbm.at[pl.ds(base + c * CHUNK, CHUNK)], idx_v.at[b])
        gathers[c] = pltpu.async_copy(
            table_hbm.at[idx_v.at[b]], rows_v.at[b], gsems[b]
        )

    start_gather(0)
    for c in range(n_chunks):
        b = c % NBUF
        if c + 1 < n_chunks:
            if c + 1 >= NBUF:
                stores[c + 1 - NBUF].wait()
            start_gather(c + 1)
        gathers[c].wait()
        stores[c] = pltpu.async_copy(
            rows_v.at[b], out_hbm.at[pl.ds(base + c * CHUNK, CHUNK)], ssems[b]
        )
    for c in range(max(0, n_chunks - NBUF), n_chunks):
        stores[c].wait()


def _embed_lookup(idx_flat, table):
    n = idx_flat.shape[0]
    mesh = plsc.VectorSubcoreMesh(core_axis_name="c", subcore_axis_name="s")
    n_blocks = (VOCAB + TBLK - 1) // TBLK
    t4 = pl.pallas_call(
        _tc_transpose_body,
        grid=(n_blocks,),
        in_specs=[pl.BlockSpec((32, TBLK), lambda j: (0, j))],
        out_specs=pl.BlockSpec((TBLK // 4, 128), lambda j: (j, 0)),
        out_shape=jax.ShapeDtypeStruct((n_blocks * TBLK // 4, 128), jnp.float32),
    )(table.T)
    t_lin = t4.reshape(n_blocks * TBLK, EMBED_DIM)
    return pl.kernel(
        _gather_body,
        out_type=jax.ShapeDtypeStruct((n, EMBED_DIM), jnp.float32),
        mesh=mesh,
        scratch_types=[
            pltpu.VMEM((NBUF, CHUNK), jnp.int32),
            pltpu.VMEM((NBUF, CHUNK, EMBED_DIM), jnp.float32),
        ]
        + [pltpu.SemaphoreType.DMA] * (2 * NBUF),
        compiler_params=pltpu.CompilerParams(use_tc_tiling_on_sc=False),
    )(idx_flat, t_lin)


def kernel(embedding_input, embedding):
    batch, hist = embedding_input.shape
    idx = embedding_input.reshape(-1).astype(jnp.int32)
    slot = (idx & ~8191) | ((idx & 2047) << 2) | ((idx >> 11) & 3)
    out = _embed_lookup(slot, embedding)
    return out.reshape(batch, hist, EMBED_DIM)


# speedup vs baseline: 2.6742x; 1.0080x over previous
import jax
import jax.numpy as jnp
from jax import lax
from jax.experimental import pallas as pl
from jax.experimental.pallas import tpu as pltpu
from jax.experimental.pallas import tpu_sc as plsc

EMBED_DIM = 32
NUM_CORES = 2
NUM_SUBCORES = 16
NUM_WORKERS = NUM_CORES * NUM_SUBCORES
VOCAB = 1000000
EBLK = 128
NFULL = VOCAB // EBLK
ETAIL = VOCAB - NFULL * EBLK

CHUNK = 1024
NBUF = 3


TBLK = 16384


def _tc_transpose_body(embT_ref, t4_ref):
    y = embT_ref[...].T
    for a in range(4):
        t4_ref[:, pl.ds(32 * a, 32)] = lax.slice(
            y, (a * (TBLK // 4), 0), ((a + 1) * (TBLK // 4), 32)
        )


def _gather_body(idx_hbm, table_hbm, out_hbm, idx_v, rows_v, *sems):
    gsems, ssems = sems[:NBUF], sems[NBUF:]
    n_chunks = idx_hbm.shape[0] // (NUM_WORKERS * CHUNK)
    wid = lax.axis_index("s") * NUM_CORES + lax.axis_index("c")
    base = wid * (n_chunks * CHUNK)
    gathers = [None] * n_chunks
    stores = [None] * n_chunks

    def start_gather(c):
        b = c % NBUF
        pltpu.sync_copy(idx_hbm.at[pl.ds(base + c * CHUNK, CHUNK)], idx_v.at[b])
        gathers[c] = pltpu.async_copy(
            table_hbm.at[idx_v.at[b]], rows_v.at[b], gsems[b]
        )

    start_gather(0)
    for c in range(n_chunks):
        b = c % NBUF
        if c + 1 < n_chunks:
            if c + 1 >= NBUF:
                stores[c + 1 - NBUF].wait()
            start_gather(c + 1)
        gathers[c].wait()
        stores[c] = pltpu.async_copy(
            rows_v.at[b], out_hbm.at[pl.ds(base + c * CHUNK, CHUNK)], ssems[b]
        )
    for c in range(max(0, n_chunks - NBUF), n_chunks):
        stores[c].wait()


def _embed_lookup(idx_flat, table):
    n = idx_flat.shape[0]
    mesh = plsc.VectorSubcoreMesh(core_axis_name="c", subcore_axis_name="s")
    n_blocks = (VOCAB + TBLK - 1) // TBLK
    t4 = pl.pallas_call(
        _tc_transpose_body,
        grid=(n_blocks,),
        in_specs=[pl.BlockSpec((32, TBLK), lambda j: (0, j))],
        out_specs=pl.BlockSpec((TBLK // 4, 128), lambda j: (j, 0)),
        out_shape=jax.ShapeDtypeStruct((n_blocks * TBLK // 4, 128), jnp.float32),
    )(table.T)
    t_lin = t4.reshape(n_blocks * TBLK, EMBED_DIM)
    return pl.kernel(
        _gather_body,
        out_type=jax.ShapeDtypeStruct((n, EMBED_DIM), jnp.float32),
        mesh=mesh,
        scratch_types=[
            pltpu.VMEM((NBUF, CHUNK), jnp.int32),
            pltpu.VMEM((NBUF, CHUNK, EMBED_DIM), jnp.float32),
        ]
        + [pltpu.SemaphoreType.DMA] * (2 * NBUF),
        compiler_params=pltpu.CompilerParams(use_tc_tiling_on_sc=False),
    )(idx_flat, t_lin)


def kernel(embedding_input, embedding):
    batch, hist = embedding_input.shape
    idx = embedding_input.reshape(-1).astype(jnp.int32)
    slot = (idx & ~16383) | ((idx & 4095) << 2) | ((idx >> 12) & 3)
    out = _embed_lookup(slot, embedding)
    return out.reshape(batch, hist, EMBED_DIM)


# speedup vs baseline: 2.6854x; 1.0042x over previous
import jax
import jax.numpy as jnp
from jax import lax
from jax.experimental import pallas as pl
from jax.experimental.pallas import tpu as pltpu
from jax.experimental.pallas import tpu_sc as plsc

EMBED_DIM = 32
NUM_CORES = 2
NUM_SUBCORES = 16
NUM_WORKERS = NUM_CORES * NUM_SUBCORES
VOCAB = 1000000
EBLK = 128
NFULL = VOCAB // EBLK
ETAIL = VOCAB - NFULL * EBLK

CHUNK = 1024
NBUF = 3


TBLK = 32768


def _tc_transpose_body(embT_ref, t4_ref):
    y = embT_ref[...].T
    for a in range(4):
        t4_ref[:, pl.ds(32 * a, 32)] = lax.slice(
            y, (a * (TBLK // 4), 0), ((a + 1) * (TBLK // 4), 32)
        )


def _gather_body(idx_hbm, table_hbm, out_hbm, idx_v, rows_v, *sems):
    gsems, ssems = sems[:NBUF], sems[NBUF:]
    n_chunks = idx_hbm.shape[0] // (NUM_WORKERS * CHUNK)
    wid = lax.axis_index("s") * NUM_CORES + lax.axis_index("c")
    base = wid * (n_chunks * CHUNK)
    gathers = [None] * n_chunks
    stores = [None] * n_chunks

    def start_gather(c):
        b = c % NBUF
        pltpu.sync_copy(idx_hbm.at[pl.ds(base + c * CHUNK, CHUNK)], idx_v.at[b])
        gathers[c] = pltpu.async_copy(
            table_hbm.at[idx_v.at[b]], rows_v.at[b], gsems[b]
        )

    start_gather(0)
    for c in range(n_chunks):
        b = c % NBUF
        if c + 1 < n_chunks:
            if c + 1 >= NBUF:
                stores[c + 1 - NBUF].wait()
            start_gather(c + 1)
        gathers[c].wait()
        stores[c] = pltpu.async_copy(
            rows_v.at[b], out_hbm.at[pl.ds(base + c * CHUNK, CHUNK)], ssems[b]
        )
    for c in range(max(0, n_chunks - NBUF), n_chunks):
        stores[c].wait()


def _embed_lookup(idx_flat, table):
    n = idx_flat.shape[0]
    mesh = plsc.VectorSubcoreMesh(core_axis_name="c", subcore_axis_name="s")
    n_blocks = (VOCAB + TBLK - 1) // TBLK
    t4 = pl.pallas_call(
        _tc_transpose_body,
        grid=(n_blocks,),
        in_specs=[pl.BlockSpec((32, TBLK), lambda j: (0, j))],
        out_specs=pl.BlockSpec((TBLK // 4, 128), lambda j: (j, 0)),
        out_shape=jax.ShapeDtypeStruct((n_blocks * TBLK // 4, 128), jnp.float32),
    )(table.T)
    t_lin = t4.reshape(n_blocks * TBLK, EMBED_DIM)
    return pl.kernel(
        _gather_body,
        out_type=jax.ShapeDtypeStruct((n, EMBED_DIM), jnp.float32),
        mesh=mesh,
        scratch_types=[
            pltpu.VMEM((NBUF, CHUNK), jnp.int32),
            pltpu.VMEM((NBUF, CHUNK, EMBED_DIM), jnp.float32),
        ]
        + [pltpu.SemaphoreType.DMA] * (2 * NBUF),
        compiler_params=pltpu.CompilerParams(use_tc_tiling_on_sc=False),
    )(idx_flat, t_lin)


def kernel(embedding_input, embedding):
    batch, hist = embedding_input.shape
    idx = embedding_input.reshape(-1).astype(jnp.int32)
    slot = (idx & ~32767) | ((idx & 8191) << 2) | ((idx >> 13) & 3)
    out = _embed_lookup(slot, embedding)
    return out.reshape(batch, hist, EMBED_DIM)


# speedup vs baseline: 2.6856x; 1.0001x over previous
import jax
import jax.numpy as jnp
from jax import lax
from jax.experimental import pallas as pl
from jax.experimental.pallas import tpu as pltpu
from jax.experimental.pallas import tpu_sc as plsc

EMBED_DIM = 32
NUM_CORES = 2
NUM_SUBCORES = 16
NUM_WORKERS = NUM_CORES * NUM_SUBCORES
VOCAB = 1000000

CHUNK = 1024
NBUF = 3

TBLK = 32768


def _tc_transpose_body(embT_ref, t4_ref):
    y = embT_ref[...].T
    for a in range(4):
        t4_ref[:, pl.ds(32 * a, 32)] = lax.slice(
            y, (a * (TBLK // 4), 0), ((a + 1) * (TBLK // 4), 32)
        )


def _gather_body(idx_hbm, table_hbm, out_hbm, idx_v, rows_v, *sems):
    gsems, ssems = sems[:NBUF], sems[NBUF:]
    n_chunks = idx_hbm.shape[0] // (NUM_WORKERS * CHUNK)
    wid = lax.axis_index("s") * NUM_CORES + lax.axis_index("c")
    base = wid * (n_chunks * CHUNK)
    gathers = [None] * n_chunks
    stores = [None] * n_chunks

    def start_gather(c):
        b = c % NBUF
        pltpu.sync_copy(idx_hbm.at[pl.ds(base + c * CHUNK, CHUNK)], idx_v.at[b])
        gathers[c] = pltpu.async_copy(
            table_hbm.at[idx_v.at[b]], rows_v.at[b], gsems[b]
        )

    start_gather(0)
    for c in range(n_chunks):
        b = c % NBUF
        if c + 1 < n_chunks:
            if c + 1 >= NBUF:
                stores[c + 1 - NBUF].wait()
            start_gather(c + 1)
        gathers[c].wait()
        stores[c] = pltpu.async_copy(
            rows_v.at[b], out_hbm.at[pl.ds(base + c * CHUNK, CHUNK)], ssems[b]
        )
    for c in range(max(0, n_chunks - NBUF), n_chunks):
        stores[c].wait()


def _embed_lookup(idx_flat, table):
    n = idx_flat.shape[0]
    mesh = plsc.VectorSubcoreMesh(core_axis_name="c", subcore_axis_name="s")
    n_blocks = (VOCAB + TBLK - 1) // TBLK
    t4 = pl.pallas_call(
        _tc_transpose_body,
        grid=(n_blocks,),
        in_specs=[pl.BlockSpec((32, TBLK), lambda j: (0, j))],
        out_specs=pl.BlockSpec((TBLK // 4, 128), lambda j: (j, 0)),
        out_shape=jax.ShapeDtypeStruct((n_blocks * TBLK // 4, 128), jnp.float32),
    )(table.T)
    t_lin = t4.reshape(n_blocks * TBLK, EMBED_DIM)
    return pl.kernel(
        _gather_body,
        out_type=jax.ShapeDtypeStruct((n, EMBED_DIM), jnp.float32),
        mesh=mesh,
        scratch_types=[
            pltpu.VMEM((NBUF, CHUNK), jnp.int32),
            pltpu.VMEM((NBUF, CHUNK, EMBED_DIM), jnp.float32),
        ]
        + [pltpu.SemaphoreType.DMA] * (2 * NBUF),
        compiler_params=pltpu.CompilerParams(use_tc_tiling_on_sc=False),
    )(idx_flat, t_lin)


def kernel(embedding_input, embedding):
    batch, hist = embedding_input.shape
    idx = embedding_input.reshape(-1).astype(jnp.int32)
    slot = (idx & ~32767) | ((idx & 8191) << 2) | ((idx >> 13) & 3)
    out = _embed_lookup(slot, embedding)
    return out.reshape(batch, hist, EMBED_DIM)
